# baseline (speedup 1.0000x reference)
import functools

import jax
import jax.numpy as jnp
from jax import lax
from jax.experimental import pallas as pl
from jax.experimental.pallas import tpu as pltpu
from jax.experimental.pallas import tpu_sc as plsc

N = 10000
E = 160000
D = 256
DH = D // 2
DE = 16

NC = 2
NS = 16
LANES = 16

HALF = N // NC
NTRASH = 8
P = 5120
ZR = P // NS
EPT = E // NS
B = 128
NGRP = EPT // LANES
NB = -(-EPT // B)
NB3 = NB + (-NB) % 3
MAXE = (NB3 + 2) * B
TAIL = EPT - (NB - 1) * B
MAXF = EPT + 4 * B
FBUF = MAXF + LANES
NGI = MAXF // LANES


def _sc_body(h2_hbm, src_hbm, dst_hbm, ef_hbm, zh_hbm,
             hagg0_hbm, hagg1_hbm, efdg_hbm,
             srcflat, locu, srcb0, srcb1, srcb2,
             locb0, locb1, locb2,
             hrows0, hrows1, hrows2, efrows, acc,
             semg0, semg1, semg2, sems0, sems1, sems2):
  cid = lax.axis_index("c")
  sid = lax.axis_index("s")
  base = sid * EPT
  zr0 = sid * ZR

  pltpu.sync_copy(zh_hbm.at[pl.ds(zr0, ZR)], acc.at[pl.ds(zr0, ZR)])

  pltpu.sync_copy(src_hbm.at[pl.ds(base, EPT)], srcflat.at[pl.ds(0, EPT)])
  pltpu.sync_copy(dst_hbm.at[pl.ds(base, EPT)], locu.at[pl.ds(0, EPT)])

  iota = lax.iota(jnp.int32, LANES)

  for g in range(EPT // LANES, MAXE // LANES):
    srcflat[pl.ds(g * LANES, LANES)] = 2 * (((g - EPT // LANES) * LANES) + iota)
    locu[pl.ds(g * LANES, LANES)] = HALF + (((g * LANES) + iota) & 63)

  lo = cid * HALF
  def prep(g, c):
    vsrc = srcflat[pl.ds(g * LANES, LANES)]
    d = locu[pl.ds(g * LANES, LANES)]
    local = d - lo
    ok = (local >= 0) & (local < HALF)
    srcflat[pl.ds(g * LANES, LANES)] = vsrc * 2
    locu[pl.ds(g * LANES, LANES)] = jnp.where(ok, local, HALF + lax.rem(d, jnp.int32(120)))
    return c
  lax.fori_loop(0, EPT // LANES, prep, 0)

  plsc.subcore_barrier()

  def vcopy(src_ref, off, dst_ref, delta):
    for k in range(B // LANES):
      dst_ref[pl.ds(k * LANES, LANES)] = (
          src_ref[pl.ds(off + k * LANES, LANES)] + delta)

  srcbs = (srcb0, srcb1, srcb2)
  locbs = (locb0, locb1, locb2)
  hrowss = (hrows0, hrows1, hrows2)
  semgs = (semg0, semg1, semg2)
  semss = (sems0, sems1, sems2)
  PADOFF = MAXE - B

  def h_pass(delta):
    vcopy(locu, PADOFF, locbs[2], 0)
    pltpu.async_copy(hrowss[2], acc.at[locbs[2]], semss[2], add=True)
    for p in (0, 1):
      vcopy(srcflat, p * B, srcbs[p], delta)
      vcopy(locu, p * B, locbs[p], 0)
      pltpu.async_copy(h2_hbm.at[srcbs[p]], hrowss[p], semgs[p])
    def outer(g, c):
      for p in range(3):
        idx = g * 3 + p
        q = (p + 2) % 3
        pltpu.make_async_copy(h2_hbm.at[srcbs[p]], hrowss[p],
                              semgs[p]).wait()
        pltpu.async_copy(hrowss[p], acc.at[locbs[p]], semss[p], add=True)
        pltpu.make_async_copy(hrowss[q], acc.at[locbs[q]], semss[q]).wait()
        off = (idx + 2) * B
        vcopy(srcflat, off, srcbs[q], delta)
        vcopy(locu, off, locbs[q], 0)
        pltpu.async_copy(h2_hbm.at[srcbs[q]], hrowss[q], semgs[q])
      return c
    lax.fori_loop(0, NB3 // 3, outer, 0)
    pltpu.make_async_copy(hrowss[2], acc.at[locbs[2]], semss[2]).wait()
    for p in (0, 1):
      pltpu.make_async_copy(h2_hbm.at[srcbs[p]], hrowss[p], semgs[p]).wait()

  s0 = jnp.where(sid == NS - 1, HALF - ZR, zr0)
  out0 = cid * HALF + s0

  h_pass(0)
  plsc.subcore_barrier()
  pltpu.sync_copy(acc.at[pl.ds(s0, ZR)], hagg0_hbm.at[pl.ds(out0, ZR)])
  plsc.subcore_barrier()
  pltpu.sync_copy(zh_hbm.at[pl.ds(zr0, ZR)], acc.at[pl.ds(zr0, ZR)])
  plsc.subcore_barrier()

  h_pass(1)
  plsc.subcore_barrier()
  pltpu.sync_copy(acc.at[pl.ds(s0, ZR)], hagg1_hbm.at[pl.ds(out0, ZR)])
  plsc.subcore_barrier()
  pltpu.sync_copy(zh_hbm.at[pl.ds(zr0, ZR)], acc.at[pl.ds(zr0, ZR)])
  plsc.subcore_barrier()

  efwides = (hrows0, hrows1)
  zeros16 = jnp.zeros((LANES,), jnp.float32)
  onehot = jnp.where(iota == 0, 1.0, 0.0).astype(jnp.float32)
  def initw(e, c):
    for p in range(2):
      efwides[p][e, pl.ds(DE, LANES)] = onehot
      for k in range(2, DH // LANES):
        efwides[p][e, pl.ds(k * LANES, LANES)] = zeros16
    return c
  lax.fori_loop(0, B, initw, 0)

  for p in range(2):
    vcopy(locu, PADOFF, locbs[p], 0)
    pltpu.async_copy(efwides[p], acc.at[locbs[p]], semss[p], add=True)

  def ring2(g, c):
    for p in range(2):
      b = g * 2 + p
      pltpu.sync_copy(ef_hbm.at[pl.ds(base + b * B, B)], efrows)
      pltpu.make_async_copy(efwides[p], acc.at[locbs[p]], semss[p]).wait()
      vcopy(locu, b * B, locbs[p], 0)
      def fill(e, c2):
        efwides[p][2 * e, pl.ds(0, DE)] = efrows[2 * e, :]
        efwides[p][2 * e + 1, pl.ds(0, DE)] = efrows[2 * e + 1, :]
        return c2
      lax.fori_loop(0, B // 2, fill, 0)
      pltpu.async_copy(efwides[p], acc.at[locbs[p]], semss[p], add=True)
    return c
  lax.fori_loop(0, (NB - 1) // 2, ring2, 0)
  pltpu.sync_copy(ef_hbm.at[pl.ds(base + (NB - 1) * B, TAIL)],
                  efrows.at[pl.ds(0, TAIL)])
  tp = (NB - 1) % 2
  pltpu.make_async_copy(efwides[tp], acc.at[locbs[tp]], semss[tp]).wait()
  vcopy(locu, (NB - 1) * B, locbs[tp], 0)
  def fill_t(e, c2):
    efwides[tp][2 * e, pl.ds(0, DE)] = efrows[2 * e, :]
    efwides[tp][2 * e + 1, pl.ds(0, DE)] = efrows[2 * e + 1, :]
    return c2
  lax.fori_loop(0, B // 2, fill_t, 0)
  pltpu.async_copy(efwides[tp], acc.at[locbs[tp]], semss[tp], add=True)
  for p in range(2):
    pltpu.make_async_copy(efwides[p], acc.at[locbs[p]], semss[p]).wait()

  plsc.subcore_barrier()
  pltpu.sync_copy(acc.at[pl.ds(s0, ZR)], efdg_hbm.at[pl.ds(out0, ZR)])


_sc_aggregate = functools.partial(
    pl.kernel,
    out_type=(
        jax.ShapeDtypeStruct((N, DH), jnp.float32),
        jax.ShapeDtypeStruct((N, DH), jnp.float32),
        jax.ShapeDtypeStruct((N, DH), jnp.float32),
    ),
    mesh=plsc.VectorSubcoreMesh(
        core_axis_name="c", subcore_axis_name="s",
        num_cores=NC, num_subcores=NS),
    scratch_types=(
        pltpu.VMEM((MAXE,), jnp.int32),
        pltpu.VMEM((MAXE,), jnp.int32),
        pltpu.VMEM((B,), jnp.int32),
        pltpu.VMEM((B,), jnp.int32),
        pltpu.VMEM((B,), jnp.int32),
        pltpu.VMEM((B,), jnp.int32),
        pltpu.VMEM((B,), jnp.int32),
        pltpu.VMEM((B,), jnp.int32),
        pltpu.VMEM((B, DH), jnp.float32),
        pltpu.VMEM((B, DH), jnp.float32),
        pltpu.VMEM((B, DH), jnp.float32),
        pltpu.VMEM((B, DE), jnp.float32),
        pltpu.VMEM_SHARED((P, DH), jnp.float32),
        pltpu.SemaphoreType.DMA,
        pltpu.SemaphoreType.DMA,
        pltpu.SemaphoreType.DMA,
        pltpu.SemaphoreType.DMA,
        pltpu.SemaphoreType.DMA,
        pltpu.SemaphoreType.DMA,
    ),
)(_sc_body)


def _tc_body(hagg0_ref, hagg1_ref, efdg_ref, w_ref, we_ref, b_ref, out_ref):
  efdg = efdg_ref[...]
  eh = jnp.dot(efdg[:, :DE], we_ref[...], preferred_element_type=jnp.float32)
  deg = efdg[:, DE:DE + 1]
  norm = lax.rsqrt(jnp.maximum(deg, 1.0))
  s0 = (hagg0_ref[...] + eh[:, :DH]) * norm
  s1 = (hagg1_ref[...] + eh[:, DH:]) * norm
  out = jnp.dot(s0, w_ref[:DH, :], preferred_element_type=jnp.float32)
  out = out + jnp.dot(s1, w_ref[DH:, :], preferred_element_type=jnp.float32)
  out_ref[...] = jnp.maximum(out + b_ref[...], 0.0)


RB = 2000


def _tc_dense(hagg0, hagg1, efdg, W, W_edge, b2):
  return pl.pallas_call(
      _tc_body,
      grid=(N // RB,),
      in_specs=[
          pl.BlockSpec((RB, DH), lambda i: (i, 0)),
          pl.BlockSpec((RB, DH), lambda i: (i, 0)),
          pl.BlockSpec((RB, DH), lambda i: (i, 0)),
          pl.BlockSpec((D, D), lambda i: (0, 0)),
          pl.BlockSpec((DE, D), lambda i: (0, 0)),
          pl.BlockSpec((1, D), lambda i: (0, 0)),
      ],
      out_specs=pl.BlockSpec((RB, D), lambda i: (i, 0)),
      out_shape=jax.ShapeDtypeStruct((N, D), jnp.float32),
  )(hagg0, hagg1, efdg, W, W_edge, b2)


def kernel(h, edge_index, edge_feat, W, W_edge, b):
  src = edge_index[0].astype(jnp.int32)
  dst = edge_index[1].astype(jnp.int32)
  h2 = h.reshape(2 * N, DH)
  zeros_h = jnp.zeros((P, DH), jnp.float32)
  hagg0, hagg1, efdg = _sc_aggregate(h2, src, dst, edge_feat, zeros_h)
  return _tc_dense(hagg0, hagg1, efdg, W, W_edge, b.reshape(1, D))


# speedup vs baseline: 1.0551x; 1.0551x over previous
import functools

import jax
import jax.numpy as jnp
from jax import lax
from jax.experimental import pallas as pl
from jax.experimental.pallas import tpu as pltpu
from jax.experimental.pallas import tpu_sc as plsc

N = 10000
E = 160000
D = 256
DH = D // 2
DE = 16

NC = 2
NS = 16
LANES = 16

HALF = N // NC
NTRASH = 8
P = 5120
ZR = P // NS
EPT = E // NS
B = 128
NGRP = EPT // LANES
NB = -(-EPT // B)
NB3 = NB + (-NB) % 3
MAXE = (NB3 + 2) * B
TAIL = EPT - (NB - 1) * B
MAXF = EPT + 4 * B
FBUF = MAXF + LANES
NGI = MAXF // LANES


def _sc_body(h2_hbm, src_hbm, dst_hbm, ef_hbm, zh_hbm,
             hagg0_hbm, hagg1_hbm, efdg_hbm,
             srcflat, locu, srcb0, srcb1, srcb2,
             locb0, locb1, locb2,
             hrows0, hrows1, hrows2, efrows, acc,
             semg0, semg1, semg2, sems0, sems1, sems2):
  cid = lax.axis_index("c")
  sid = lax.axis_index("s")
  base = sid * EPT
  zr0 = sid * ZR

  pltpu.sync_copy(zh_hbm.at[pl.ds(zr0, ZR)], acc.at[pl.ds(zr0, ZR)])

  pltpu.sync_copy(src_hbm.at[pl.ds(base, EPT)], srcflat.at[pl.ds(0, EPT)])
  pltpu.sync_copy(dst_hbm.at[pl.ds(base, EPT)], locu.at[pl.ds(0, EPT)])

  iota = lax.iota(jnp.int32, LANES)

  for g in range(EPT // LANES, MAXE // LANES):
    srcflat[pl.ds(g * LANES, LANES)] = 2 * (((g - EPT // LANES) * LANES) + iota)
    locu[pl.ds(g * LANES, LANES)] = HALF + (((g * LANES) + iota) & 63)

  lo = cid * HALF
  def prep(g, c):
    vsrc = srcflat[pl.ds(g * LANES, LANES)]
    d = locu[pl.ds(g * LANES, LANES)]
    local = d - lo
    ok = (local >= 0) & (local < HALF)
    srcflat[pl.ds(g * LANES, LANES)] = vsrc * 2
    locu[pl.ds(g * LANES, LANES)] = jnp.where(ok, local, HALF + (d & 63))
    return c
  lax.fori_loop(0, EPT // LANES, prep, 0)

  plsc.subcore_barrier()

  def vcopy(src_ref, off, dst_ref, delta):
    for k in range(B // LANES):
      dst_ref[pl.ds(k * LANES, LANES)] = (
          src_ref[pl.ds(off + k * LANES, LANES)] + delta)

  srcbs = (srcb0, srcb1, srcb2)
  locbs = (locb0, locb1, locb2)
  hrowss = (hrows0, hrows1, hrows2)
  semgs = (semg0, semg1, semg2)
  semss = (sems0, sems1, sems2)
  PADOFF = MAXE - B

  def h_pass(delta):
    vcopy(locu, PADOFF, locbs[2], 0)
    pltpu.async_copy(hrowss[2], acc.at[locbs[2]], semss[2], add=True)
    for p in (0, 1):
      vcopy(srcflat, p * B, srcbs[p], delta)
      vcopy(locu, p * B, locbs[p], 0)
      pltpu.async_copy(h2_hbm.at[srcbs[p]], hrowss[p], semgs[p])
    def outer(g, c):
      for p in range(3):
        idx = g * 3 + p
        q = (p + 2) % 3
        pltpu.make_async_copy(h2_hbm.at[srcbs[p]], hrowss[p],
                              semgs[p]).wait()
        pltpu.async_copy(hrowss[p], acc.at[locbs[p]], semss[p], add=True)
        pltpu.make_async_copy(hrowss[q], acc.at[locbs[q]], semss[q]).wait()
        off = (idx + 2) * B
        vcopy(srcflat, off, srcbs[q], delta)
        vcopy(locu, off, locbs[q], 0)
        pltpu.async_copy(h2_hbm.at[srcbs[q]], hrowss[q], semgs[q])
      return c
    lax.fori_loop(0, NB3 // 3, outer, 0)
    pltpu.make_async_copy(hrowss[2], acc.at[locbs[2]], semss[2]).wait()
    for p in (0, 1):
      pltpu.make_async_copy(h2_hbm.at[srcbs[p]], hrowss[p], semgs[p]).wait()

  s0 = jnp.where(sid == NS - 1, HALF - ZR, zr0)
  out0 = cid * HALF + s0

  h_pass(0)
  plsc.subcore_barrier()
  pltpu.sync_copy(acc.at[pl.ds(s0, ZR)], hagg0_hbm.at[pl.ds(out0, ZR)])
  plsc.subcore_barrier()
  pltpu.sync_copy(zh_hbm.at[pl.ds(zr0, ZR)], acc.at[pl.ds(zr0, ZR)])
  plsc.subcore_barrier()

  h_pass(1)
  plsc.subcore_barrier()
  pltpu.sync_copy(acc.at[pl.ds(s0, ZR)], hagg1_hbm.at[pl.ds(out0, ZR)])
  plsc.subcore_barrier()
  pltpu.sync_copy(zh_hbm.at[pl.ds(zr0, ZR)], acc.at[pl.ds(zr0, ZR)])
  plsc.subcore_barrier()

  efwides = (hrows0, hrows1)
  zeros16 = jnp.zeros((LANES,), jnp.float32)
  onehot = jnp.where(iota == 0, 1.0, 0.0).astype(jnp.float32)
  def initw(e, c):
    for p in range(2):
      efwides[p][e, pl.ds(DE, LANES)] = onehot
      for k in range(2, DH // LANES):
        efwides[p][e, pl.ds(k * LANES, LANES)] = zeros16
    return c
  lax.fori_loop(0, B, initw, 0)

  for p in range(2):
    vcopy(locu, PADOFF, locbs[p], 0)
    pltpu.async_copy(efwides[p], acc.at[locbs[p]], semss[p], add=True)

  def ring2(g, c):
    for p in range(2):
      b = g * 2 + p
      pltpu.sync_copy(ef_hbm.at[pl.ds(base + b * B, B)], efrows)
      pltpu.make_async_copy(efwides[p], acc.at[locbs[p]], semss[p]).wait()
      vcopy(locu, b * B, locbs[p], 0)
      def fill(e, c2):
        efwides[p][2 * e, pl.ds(0, DE)] = efrows[2 * e, :]
        efwides[p][2 * e + 1, pl.ds(0, DE)] = efrows[2 * e + 1, :]
        return c2
      lax.fori_loop(0, B // 2, fill, 0)
      pltpu.async_copy(efwides[p], acc.at[locbs[p]], semss[p], add=True)
    return c
  lax.fori_loop(0, (NB - 1) // 2, ring2, 0)
  pltpu.sync_copy(ef_hbm.at[pl.ds(base + (NB - 1) * B, TAIL)],
                  efrows.at[pl.ds(0, TAIL)])
  tp = (NB - 1) % 2
  pltpu.make_async_copy(efwides[tp], acc.at[locbs[tp]], semss[tp]).wait()
  vcopy(locu, (NB - 1) * B, locbs[tp], 0)
  def fill_t(e, c2):
    efwides[tp][2 * e, pl.ds(0, DE)] = efrows[2 * e, :]
    efwides[tp][2 * e + 1, pl.ds(0, DE)] = efrows[2 * e + 1, :]
    return c2
  lax.fori_loop(0, B // 2, fill_t, 0)
  pltpu.async_copy(efwides[tp], acc.at[locbs[tp]], semss[tp], add=True)
  for p in range(2):
    pltpu.make_async_copy(efwides[p], acc.at[locbs[p]], semss[p]).wait()

  plsc.subcore_barrier()
  pltpu.sync_copy(acc.at[pl.ds(s0, ZR)], efdg_hbm.at[pl.ds(out0, ZR)])


_sc_aggregate = functools.partial(
    pl.kernel,
    out_type=(
        jax.ShapeDtypeStruct((N, DH), jnp.float32),
        jax.ShapeDtypeStruct((N, DH), jnp.float32),
        jax.ShapeDtypeStruct((N, DH), jnp.float32),
    ),
    mesh=plsc.VectorSubcoreMesh(
        core_axis_name="c", subcore_axis_name="s",
        num_cores=NC, num_subcores=NS),
    scratch_types=(
        pltpu.VMEM((MAXE,), jnp.int32),
        pltpu.VMEM((MAXE,), jnp.int32),
        pltpu.VMEM((B,), jnp.int32),
        pltpu.VMEM((B,), jnp.int32),
        pltpu.VMEM((B,), jnp.int32),
        pltpu.VMEM((B,), jnp.int32),
        pltpu.VMEM((B,), jnp.int32),
        pltpu.VMEM((B,), jnp.int32),
        pltpu.VMEM((B, DH), jnp.float32),
        pltpu.VMEM((B, DH), jnp.float32),
        pltpu.VMEM((B, DH), jnp.float32),
        pltpu.VMEM((B, DE), jnp.float32),
        pltpu.VMEM_SHARED((P, DH), jnp.float32),
        pltpu.SemaphoreType.DMA,
        pltpu.SemaphoreType.DMA,
        pltpu.SemaphoreType.DMA,
        pltpu.SemaphoreType.DMA,
        pltpu.SemaphoreType.DMA,
        pltpu.SemaphoreType.DMA,
    ),
)(_sc_body)


def _tc_body(hagg0_ref, hagg1_ref, efdg_ref, w_ref, we_ref, b_ref, out_ref):
  efdg = efdg_ref[...]
  eh = jnp.dot(efdg[:, :DE], we_ref[...], preferred_element_type=jnp.float32)
  deg = efdg[:, DE:DE + 1]
  norm = lax.rsqrt(jnp.maximum(deg, 1.0))
  s0 = (hagg0_ref[...] + eh[:, :DH]) * norm
  s1 = (hagg1_ref[...] + eh[:, DH:]) * norm
  out = jnp.dot(s0, w_ref[:DH, :], preferred_element_type=jnp.float32)
  out = out + jnp.dot(s1, w_ref[DH:, :], preferred_element_type=jnp.float32)
  out_ref[...] = jnp.maximum(out + b_ref[...], 0.0)


RB = 2000


def _tc_dense(hagg0, hagg1, efdg, W, W_edge, b2):
  return pl.pallas_call(
      _tc_body,
      grid=(N // RB,),
      in_specs=[
          pl.BlockSpec((RB, DH), lambda i: (i, 0)),
          pl.BlockSpec((RB, DH), lambda i: (i, 0)),
          pl.BlockSpec((RB, DH), lambda i: (i, 0)),
          pl.BlockSpec((D, D), lambda i: (0, 0)),
          pl.BlockSpec((DE, D), lambda i: (0, 0)),
          pl.BlockSpec((1, D), lambda i: (0, 0)),
      ],
      out_specs=pl.BlockSpec((RB, D), lambda i: (i, 0)),
      out_shape=jax.ShapeDtypeStruct((N, D), jnp.float32),
  )(hagg0, hagg1, efdg, W, W_edge, b2)


def kernel(h, edge_index, edge_feat, W, W_edge, b):
  src = edge_index[0].astype(jnp.int32)
  dst = edge_index[1].astype(jnp.int32)
  h2 = h.reshape(2 * N, DH)
  zeros_h = jnp.zeros((P, DH), jnp.float32)
  hagg0, hagg1, efdg = _sc_aggregate(h2, src, dst, edge_feat, zeros_h)
  return _tc_dense(hagg0, hagg1, efdg, W, W_edge, b.reshape(1, D))
